# baseline (speedup 1.0000x reference)
import functools

import numpy as np

import jax
import jax.numpy as jnp
from jax import lax
from jax.experimental import pallas as pl
from jax.experimental.pallas import tpu as pltpu
from jax.experimental.pallas import tpu_sc as plsc

_B, _D, _H, _W = 4, 64, 256, 256
_SD, _SH, _SW = 16, 64, 64
_NHOLES = 4
_DBLK = 16


def _hole_offsets_np():
    base_key = jax.random.key(42)
    rows = []
    for hole_idx in range(1, _NHOLES + 1):
        k = jax.random.fold_in(base_key, hole_idx)
        kz, ky, kx = jax.random.split(k, 3)
        rows.append(np.stack([
            np.asarray(jax.random.randint(kz, (_B,), 0, _D - _SD + 1)),
            np.asarray(jax.random.randint(ky, (_B,), 0, _H - _SH + 1)),
            np.asarray(jax.random.randint(kx, (_B,), 0, _W - _SW + 1)),
        ]))
    return np.stack(rows).astype(np.int32)


def _hole_offsets_concrete():
    try:
        cpu = jax.local_devices(backend="cpu")[0]
        with jax.default_device(cpu):
            return _hole_offsets_np()
    except RuntimeError:
        return _hole_offsets_np()


_OFFS = _hole_offsets_concrete()


def _float_kernel(offs_ref, vol_ref, gm_ref, vol_out, gm_out):
    b = pl.program_id(0)
    z0 = pl.program_id(1) * _DBLK
    ziota = lax.broadcasted_iota(jnp.int32, (1, _DBLK, 1, 1), 1) + z0
    yiota = lax.broadcasted_iota(jnp.int32, (1, 1, _H, _W), 2)
    xiota = lax.broadcasted_iota(jnp.int32, (1, 1, _H, _W), 3)
    mask = None
    for h in range(_NHOLES):
        bz = offs_ref[h, 0, b]
        by = offs_ref[h, 1, b]
        bx = offs_ref[h, 2, b]
        zm = (ziota >= bz) & (ziota < bz + _SD)
        ym = (yiota >= by) & (yiota < by + _SH)
        xm = (xiota >= bx) & (xiota < bx + _SW)
        m = zm & (ym & xm)
        mask = m if mask is None else mask | m
    vol_out[...] = jnp.where(mask, jnp.float32(0.0), vol_ref[...])
    gm_out[...] = jnp.where(mask, jnp.float32(2.0), gm_ref[...])


def _float_call(volume, gt_mask):
    offs = jnp.asarray(_OFFS)
    grid = (_B, _D // _DBLK)
    blk = (1, _DBLK, _H, _W)
    data_spec = pl.BlockSpec(blk, lambda b, d, offs: (b, d, 0, 0))
    out_shapes = (
        jax.ShapeDtypeStruct(volume.shape, volume.dtype),
        jax.ShapeDtypeStruct(gt_mask.shape, gt_mask.dtype),
    )
    grid_spec = pltpu.PrefetchScalarGridSpec(
        num_scalar_prefetch=1,
        grid=grid,
        in_specs=[data_spec, data_spec],
        out_specs=[data_spec, data_spec],
    )
    return pl.pallas_call(
        _float_kernel,
        grid_spec=grid_spec,
        out_shape=out_shapes,
        compiler_params=pltpu.CompilerParams(
            vmem_limit_bytes=110 * 1024 * 1024,
        ),
    )(offs, volume, gt_mask)


_NW = 32
_SLICES = _B * _D
_PER_W = _SLICES // _NW
_SLICE_WORDS = _H * _W // 4
_ROW_WORDS = _W // 4


def _hole_word_groups(bx):
    groups = []
    for g in range(_ROW_WORDS // 16):
        lo, hi = 64 * g, 64 * g + 64
        if lo < bx + _SW and hi > bx:
            groups.append(g)
    return groups


_FSLICE_WORDS = _H * _W


def _mask_sc_kernel(skel_hbm, out_hbm, skel_out_hbm, buf, fbuf):
    cid = lax.axis_index("c")
    sid = lax.axis_index("s")
    wid = sid * 2 + cid

    zero16 = lax.broadcast_in_dim(wid * 0, (16,), ())
    wiota = lax.iota(jnp.int32, 16)
    two16 = lax.broadcast_in_dim(
        lax.convert_element_type(wid * 0, jnp.float32) + 2.0, (16,), ())

    def _skel_body(i, c):
        s = wid * _PER_W + i
        b = s // _D
        z = lax.rem(s, _D)
        pltpu.sync_copy(skel_hbm.at[pl.ds(s * _FSLICE_WORDS, _FSLICE_WORDS)],
                        fbuf)
        for bb in range(_B):
            for h in range(_NHOLES):
                bz = int(_OFFS[h, 0, bb])
                by = int(_OFFS[h, 1, bb])
                bx = int(_OFFS[h, 2, bb])
                pred = (b == bb) & (z >= bz) & (z < bz + _SD)

                @pl.when(pred)
                def _():
                    g_lo = bx // 16
                    g_hi = (bx + _SW - 1) // 16
                    def _row(r, rc):
                        base = (by + r) * _W
                        for g in range(g_lo, g_hi + 1):
                            off = base + g * 16
                            if g * 16 >= bx and (g + 1) * 16 <= bx + _SW:
                                fbuf[pl.ds(off, 16)] = two16
                            else:
                                w0 = g * 16 + wiota
                                on = (w0 >= bx) & (w0 < bx + _SW)
                                fbuf[pl.ds(off, 16)] = jnp.where(
                                    on, two16, fbuf[pl.ds(off, 16)])
                        return rc
                    lax.fori_loop(0, _SH, _row, 0)
        pltpu.sync_copy(fbuf,
                        skel_out_hbm.at[pl.ds(s * _FSLICE_WORDS,
                                              _FSLICE_WORDS)])
        return c

    lax.fori_loop(0, _PER_W, _skel_body, 0)

    def _pattern(g, bx):
        w0 = g * 16 + wiota
        acc = zero16
        for p in range(4):
            bidx = w0 * 4 + p
            on = (bidx >= bx) & (bidx < bx + _SW)
            acc = acc | jnp.where(on, jnp.int32(1 << (8 * p)), jnp.int32(0))
        return acc

    def _clear(i, c):
        buf[pl.ds(i * 16, 16)] = zero16
        return c
    lax.fori_loop(0, _SLICE_WORDS // 16, _clear, 0)

    def _slice_body(i, c):
        s = wid * _PER_W + i
        b = s // _D
        z = lax.rem(s, _D)

        def _paint(erase):
            for bb in range(_B):
                for h in range(_NHOLES):
                    bz = int(_OFFS[h, 0, bb])
                    by = int(_OFFS[h, 1, bb])
                    bx = int(_OFFS[h, 2, bb])
                    groups = _hole_word_groups(bx)
                    pred = (b == bb) & (z >= bz) & (z < bz + _SD)

                    @pl.when(pred)
                    def _():
                        pats = None if erase else [
                            _pattern(g, bx) for g in groups]

                        def _row(r, rc):
                            base = (by + r) * _ROW_WORDS
                            for gi, g in enumerate(groups):
                                off = base + g * 16
                                if erase:
                                    buf[pl.ds(off, 16)] = zero16
                                else:
                                    buf[pl.ds(off, 16)] = (
                                        buf[pl.ds(off, 16)] | pats[gi])
                            return rc
                        lax.fori_loop(0, _SH, _row, 0)

        _paint(erase=False)
        pltpu.sync_copy(buf, out_hbm.at[pl.ds(s * _SLICE_WORDS, _SLICE_WORDS)])
        _paint(erase=True)
        return c

    lax.fori_loop(0, _PER_W, _slice_body, 0)


def _sc_call(gt_skel):
    mesh = plsc.VectorSubcoreMesh(core_axis_name="c", subcore_axis_name="s")
    fn = functools.partial(
        pl.kernel,
        mesh=mesh,
        out_type=(
            jax.ShapeDtypeStruct((_B * _D * _H * _W // 4,), jnp.int32),
            jax.ShapeDtypeStruct((_B * _D * _H * _W,), jnp.float32),
        ),
        scratch_types=[pltpu.VMEM((_SLICE_WORDS,), jnp.int32),
                       pltpu.VMEM((_FSLICE_WORDS,), jnp.float32)],
    )(_mask_sc_kernel)
    return fn(gt_skel.reshape(-1))


@jax.jit
def kernel(volume, gt_mask, gt_skel):
    vol, gm = _float_call(volume, gt_mask)
    msk_words, gs = _sc_call(gt_skel)
    gs = gs.reshape(_B, _D, _H, _W)
    msk = lax.bitcast_convert_type(msk_words, jnp.int8)
    msk = msk.reshape(_B, _D, _H, _W).astype(jnp.bool_)
    return vol, gm, gs, msk


# speedup vs baseline: 1.4162x; 1.4162x over previous
import functools

import jax
import jax.numpy as jnp
from jax.experimental import pallas as pl
from jax.experimental.pallas import tpu as pltpu

_B, _D, _H, _W = 4, 64, 256, 256
_SD, _SH, _SW = 16, 64, 64
_NHOLES = 4
_DBLK = 16


def _hole_offsets():
    base_key = jax.random.key(42)
    rows = []
    for hole_idx in range(1, _NHOLES + 1):
        k = jax.random.fold_in(base_key, hole_idx)
        kz, ky, kx = jax.random.split(k, 3)
        bz = jax.random.randint(kz, (_B,), 0, _D - _SD + 1)
        by = jax.random.randint(ky, (_B,), 0, _H - _SH + 1)
        bx = jax.random.randint(kx, (_B,), 0, _W - _SW + 1)
        rows.append(jnp.stack([bz, by, bx]))
    return jnp.stack(rows).astype(jnp.int32)


def _cutout_kernel(offs_ref, vol_ref, gm_ref, gs_ref,
                   vol_out, gm_out, gs_out, msk_out):
    b = pl.program_id(0)
    dblk = pl.program_id(1)
    z0 = dblk * _DBLK
    ziota = jax.lax.broadcasted_iota(jnp.int32, (1, _DBLK, 1, 1), 1) + z0
    yiota = jax.lax.broadcasted_iota(jnp.int32, (1, 1, _H, _W), 2)
    xiota = jax.lax.broadcasted_iota(jnp.int32, (1, 1, _H, _W), 3)
    zmask = jnp.zeros((1, _DBLK, 1, 1), jnp.bool_)
    yxmask = jnp.zeros((1, 1, _H, _W), jnp.bool_)
    mask = jnp.zeros((1, _DBLK, _H, _W), jnp.bool_)
    for h in range(_NHOLES):
        bz = offs_ref[h, 0, b]
        by = offs_ref[h, 1, b]
        bx = offs_ref[h, 2, b]
        zm = (ziota >= bz) & (ziota < bz + _SD)
        ym = (yiota >= by) & (yiota < by + _SH)
        xm = (xiota >= bx) & (xiota < bx + _SW)
        mask = mask | (zm & (ym & xm))
    vol_out[...] = jnp.where(mask, jnp.float32(0.0), vol_ref[...])
    gm_out[...] = jnp.where(mask, jnp.float32(2.0), gm_ref[...])
    gs_out[...] = jnp.where(mask, jnp.float32(2.0), gs_ref[...])
    msk_out[...] = mask.astype(jnp.int8)


@jax.jit
def kernel(volume, gt_mask, gt_skel):
    offs = _hole_offsets()
    grid = (_B, _D // _DBLK)
    blk = (1, _DBLK, _H, _W)
    data_spec = pl.BlockSpec(blk, lambda b, d, offs: (b, d, 0, 0))
    out_shapes = (
        jax.ShapeDtypeStruct(volume.shape, volume.dtype),
        jax.ShapeDtypeStruct(gt_mask.shape, gt_mask.dtype),
        jax.ShapeDtypeStruct(gt_skel.shape, gt_skel.dtype),
        jax.ShapeDtypeStruct(volume.shape, jnp.int8),
    )
    grid_spec = pltpu.PrefetchScalarGridSpec(
        num_scalar_prefetch=1,
        grid=grid,
        in_specs=[data_spec, data_spec, data_spec],
        out_specs=[data_spec, data_spec, data_spec, data_spec],
    )
    vol, gm, gs, msk = pl.pallas_call(
        _cutout_kernel,
        grid_spec=grid_spec,
        out_shape=out_shapes,
        compiler_params=pltpu.CompilerParams(
            vmem_limit_bytes=110 * 1024 * 1024,
        ),
    )(offs, volume, gt_mask, gt_skel)
    return vol, gm, gs, msk.astype(jnp.bool_)


# speedup vs baseline: 2.2041x; 1.5564x over previous
import numpy as np

import jax
import jax.numpy as jnp
from jax import lax
from jax.experimental import pallas as pl
from jax.experimental.pallas import tpu as pltpu

_B, _D, _H, _W = 4, 64, 256, 256
_SD, _SH, _SW = 16, 64, 64
_NHOLES = 4
_DBLK = 16


def _hole_offsets_np():
    base_key = jax.random.key(42)
    rows = []
    for hole_idx in range(1, _NHOLES + 1):
        k = jax.random.fold_in(base_key, hole_idx)
        kz, ky, kx = jax.random.split(k, 3)
        rows.append(np.stack([
            np.asarray(jax.random.randint(kz, (_B,), 0, _D - _SD + 1)),
            np.asarray(jax.random.randint(ky, (_B,), 0, _H - _SH + 1)),
            np.asarray(jax.random.randint(kx, (_B,), 0, _W - _SW + 1)),
        ]))
    return np.stack(rows).astype(np.int32)


def _hole_offsets_concrete():
    try:
        cpu = jax.local_devices(backend="cpu")[0]
        with jax.default_device(cpu):
            return _hole_offsets_np()
    except RuntimeError:
        return _hole_offsets_np()


_OFFS = _hole_offsets_concrete()


def _block_mask(offs_ref):
    b = pl.program_id(0)
    z0 = pl.program_id(1) * _DBLK
    ziota = lax.broadcasted_iota(jnp.int32, (1, _DBLK, 1, 1), 1) + z0
    yiota = lax.broadcasted_iota(jnp.int32, (1, 1, _H, _W), 2)
    xiota = lax.broadcasted_iota(jnp.int32, (1, 1, _H, _W), 3)
    mask = None
    for h in range(_NHOLES):
        bz = offs_ref[h, 0, b]
        by = offs_ref[h, 1, b]
        bx = offs_ref[h, 2, b]
        zm = (ziota >= bz) & (ziota < bz + _SD)
        ym = (yiota >= by) & (yiota < by + _SH)
        xm = (xiota >= bx) & (xiota < bx + _SW)
        m = zm & (ym & xm)
        mask = m if mask is None else mask | m
    return mask


def _vol_gm_kernel(offs_ref, vol_ref, gm_ref, vol_out, gm_out):
    mask = _block_mask(offs_ref)
    vol_out[...] = jnp.where(mask, jnp.float32(0.0), vol_ref[...])
    gm_out[...] = jnp.where(mask, jnp.float32(2.0), gm_ref[...])


def _gs_msk_kernel(offs_ref, gs_ref, gs_out, msk_out):
    mask = _block_mask(offs_ref)
    gs_out[...] = jnp.where(mask, jnp.float32(2.0), gs_ref[...])
    msk_out[...] = mask.astype(jnp.int8)


def _call(body, n_in, n_out, inputs, out_shapes):
    offs = jnp.asarray(_OFFS)
    grid = (_B, _D // _DBLK)
    blk = (1, _DBLK, _H, _W)
    data_spec = pl.BlockSpec(blk, lambda b, d, offs: (b, d, 0, 0))
    grid_spec = pltpu.PrefetchScalarGridSpec(
        num_scalar_prefetch=1,
        grid=grid,
        in_specs=[data_spec] * n_in,
        out_specs=[data_spec] * n_out,
    )
    return pl.pallas_call(
        body,
        grid_spec=grid_spec,
        out_shape=out_shapes,
        compiler_params=pltpu.CompilerParams(
            vmem_limit_bytes=110 * 1024 * 1024,
        ),
    )(offs, *inputs)


@jax.jit
def kernel(volume, gt_mask, gt_skel):
    f32s = jax.ShapeDtypeStruct(volume.shape, jnp.float32)
    i8s = jax.ShapeDtypeStruct(volume.shape, jnp.int8)
    vol, gm = _call(_vol_gm_kernel, 2, 2, (volume, gt_mask), (f32s, f32s))
    gs, msk = _call(_gs_msk_kernel, 1, 2, (gt_skel,), (f32s, i8s))
    return vol, gm, gs, msk.astype(jnp.bool_)
